# baseline (speedup 1.0000x reference)
import functools

import jax
import jax.numpy as jnp
from jax import lax
from jax.experimental import pallas as pl
from jax.experimental.pallas import tpu as pltpu
import jax.experimental.pallas.tpu_sc as plsc

NC = 2
NS = 16
WIN = 80


def _mesh():
    return plsc.VectorSubcoreMesh(core_axis_name="c", subcore_axis_name="s",
                                  num_cores=NC, num_subcores=NS)


@functools.lru_cache(maxsize=None)
def _make_degree(n_nodes, n_edges):
    e_per_w = n_edges // (NC * NS)
    nwin = e_per_w // WIN
    chunk = -(-n_nodes // (NS * 128)) * 128
    n_pad = NS * chunk

    @functools.partial(
        pl.kernel,
        mesh=_mesh(),
        out_type=jax.ShapeDtypeStruct((NC, n_pad), jnp.float32),
        scratch_types=[
            pltpu.VMEM((nwin, WIN), jnp.int32),
            pltpu.VMEM((WIN,), jnp.float32),
            pltpu.VMEM((chunk,), jnp.float32),
            pltpu.VMEM_SHARED((n_pad,), jnp.float32),
            pltpu.SemaphoreType.DMA,
        ],
    )
    def deg_kernel(src_hbm, zeros_hbm, out_hbm, idx_v, ones_v, z_v, acc, *sem_r):
        c = lax.axis_index("c")
        s = lax.axis_index("s")
        w = c * NS + s
        for i in range(WIN // 16):
            ones_v[pl.ds(16 * i, 16)] = jnp.ones((16,), jnp.float32)
        pltpu.sync_copy(zeros_hbm, z_v)
        pltpu.sync_copy(src_hbm.at[w], idx_v)
        pltpu.sync_copy(z_v, acc.at[pl.ds(s * chunk, chunk)])
        plsc.subcore_barrier()

        K = 4

        def body(j, carry):
            pltpu.async_copy(ones_v, acc.at[idx_v.at[j]], sem_r[0], add=True)
            @pl.when(j >= K - 1)
            def _():
                pltpu.make_async_copy(ones_v, acc.at[idx_v.at[j]],
                                      sem_r[0]).wait()
            return carry

        lax.fori_loop(0, nwin, body, 0)
        for _ in range(K - 1):
            pltpu.make_async_copy(ones_v, acc.at[idx_v.at[0]], sem_r[0]).wait()
        plsc.subcore_barrier()
        pltpu.sync_copy(acc.at[pl.ds(s * chunk, chunk)],
                        out_hbm.at[c, pl.ds(s * chunk, chunk)])

    return deg_kernel


@functools.lru_cache(maxsize=None)
def _make_spmm(n_nodes, n_edges, n_t, n_c):
    e_per_w = n_edges // (NC * NS)
    nwin = e_per_w // WIN
    nbuf = 3 if n_c >= 128 else 5
    rows_per_s = n_nodes // NS
    n_fori = (nwin - nbuf - (nbuf - 1)) // nbuf
    rest = list(range(nbuf + n_fori * nbuf, nwin))

    stage_tbl = n_c < 128

    @functools.partial(
        pl.kernel,
        mesh=_mesh(),
        compiler_params=pltpu.CompilerParams(use_tc_tiling_on_sc=False),
        out_type=jax.ShapeDtypeStruct((n_t, NC, n_nodes, n_c), jnp.float32),
        scratch_types=(
            [pltpu.VMEM((nwin, WIN), jnp.int32)] * 2
            + [pltpu.VMEM((WIN, n_c), jnp.float32)] * nbuf
            + [pltpu.VMEM_SHARED((n_nodes, n_c), jnp.float32)]
            * (2 if stage_tbl else 1)
            + [pltpu.SemaphoreType.DMA] * (2 * nbuf)
        ),
    )
    def spmm_kernel(*args):
        tbls = args[:n_t]
        src_hbm, dst_hbm, zeros_hbm, out_hbm = args[n_t:n_t + 4]
        sidx, didx = args[n_t + 4:n_t + 6]
        bufs = args[n_t + 6:n_t + 6 + nbuf]
        n_sh = 2 if stage_tbl else 1
        acc = args[n_t + 6 + nbuf]
        tbl_sp = args[n_t + 7 + nbuf] if stage_tbl else None
        sem_g = args[n_t + 6 + nbuf + n_sh:n_t + 6 + nbuf + n_sh + nbuf]
        sem_s = args[n_t + 6 + nbuf + n_sh + nbuf:]
        c = lax.axis_index("c")
        s = lax.axis_index("s")
        w = c * NS + s
        rps = rows_per_s
        pltpu.sync_copy(src_hbm.at[w], sidx)
        pltpu.sync_copy(dst_hbm.at[w], didx)

        for t in range(n_t):
            if stage_tbl:
                pltpu.sync_copy(tbls[t].at[pl.ds(s * rps, rps), :],
                                tbl_sp.at[pl.ds(s * rps, rps), :])
                tbl = tbl_sp
            else:
                tbl = tbls[t]

            def g_issue(j, b):
                pltpu.async_copy(tbl.at[sidx.at[j]], bufs[b], sem_g[b])

            def g_wait(j, b):
                pltpu.make_async_copy(tbl.at[sidx.at[j]], bufs[b],
                                      sem_g[b]).wait()

            def s_issue(j, b):
                pltpu.async_copy(bufs[b], acc.at[didx.at[j]], sem_s[b],
                                 add=True)

            def s_wait(j, b):
                pltpu.make_async_copy(bufs[b], acc.at[didx.at[j]],
                                      sem_s[b]).wait()

            def step(j, b, full, first=False):
                g_wait(j, b)
                s_issue(j, b)
                if full:
                    bn = (b + nbuf - 1) % nbuf
                    if not first:
                        s_wait(j - 1, bn)
                    g_issue(j + nbuf - 1, bn)

            pltpu.sync_copy(
                zeros_hbm, acc.at[pl.ds(s * rows_per_s, rows_per_s), :])
            plsc.subcore_barrier()

            for b in range(nbuf - 1):
                g_issue(b, b)
            for j in range(nbuf):
                step(j, j % nbuf, full=True, first=(j == 0))

            def group(q, carry):
                for b in range(nbuf):
                    step(q * nbuf + b, b, full=True)
                return carry

            lax.fori_loop(1, 1 + n_fori, group, 0)
            for j in rest:
                step(j, j % nbuf, full=(j <= nwin - nbuf))
            for b in range(nbuf):
                s_wait(nwin - nbuf + b, (nwin - nbuf + b) % nbuf)
            plsc.subcore_barrier()
            pltpu.sync_copy(
                acc.at[pl.ds(s * rows_per_s, rows_per_s), :],
                out_hbm.at[t, c, pl.ds(s * rows_per_s, rows_per_s), :])
            plsc.subcore_barrier()

    return spmm_kernel


_RB = 2000


def _dinv_from_parts(parts):
    deg = parts[:, 0] + parts[:, 1]
    return jnp.where(deg > 0, lax.rsqrt(deg), 0.0)


def _tc_prep(x, deg_parts):
    T, n, C = x.shape
    grid = (T, n // _RB)

    def body(x_ref, dp_ref, xs_ref):
        dinv = _dinv_from_parts(dp_ref[...])
        xs_ref[0] = x_ref[0] * dinv[:, None]

    return pl.pallas_call(
        body,
        grid=grid,
        in_specs=[
            pl.BlockSpec((1, _RB, C), lambda t, r: (t, r, 0)),
            pl.BlockSpec((_RB, 2), lambda t, r: (r, 0)),
        ],
        out_specs=pl.BlockSpec((1, _RB, C), lambda t, r: (t, r, 0)),
        out_shape=jax.ShapeDtypeStruct((T, n, C), jnp.float32),
    )(x, deg_parts)


def _tc_xpre(x, sx, deg_parts, w0, w1, bc, bhz, bhh, hid):
    T, n, C = x.shape
    G = w0.shape[1]
    grid = (n // _RB, T)

    def body(x_ref, sx_ref, dp_ref, w0_ref, w1_ref, bc_ref, bhz_ref, bhh_ref,
             xzr_ref, xh_ref, h_ref, hs_ref):
        t = pl.program_id(1)
        dinv = _dinv_from_parts(dp_ref[...])
        lx = (sx_ref[0, 0] + sx_ref[0, 1]) * (-dinv[:, None])
        pre = (jnp.dot(x_ref[0], w0_ref[...], preferred_element_type=jnp.float32)
               + jnp.dot(lx, w1_ref[...], preferred_element_type=jnp.float32)
               + bc_ref[...])
        xzr_ref[0] = pre[:, : 2 * hid]
        xh_ref[0] = pre[:, 2 * hid:]

        @pl.when(t == 0)
        def _():
            z = jax.nn.sigmoid(pre[:, :hid] + bhz_ref[...])
            ht = jnp.tanh(pre[:, 2 * hid:] + bhh_ref[...])
            h = (1.0 - z) * ht
            h_ref[...] = h
            hs_ref[...] = h * dinv[:, None]

    return pl.pallas_call(
        body,
        grid=grid,
        in_specs=[
            pl.BlockSpec((1, _RB, C), lambda r, t: (t, r, 0)),
            pl.BlockSpec((1, 2, _RB, C), lambda r, t: (t, 0, r, 0)),
            pl.BlockSpec((_RB, 2), lambda r, t: (r, 0)),
            pl.BlockSpec((C, G), lambda r, t: (0, 0)),
            pl.BlockSpec((C, G), lambda r, t: (0, 0)),
            pl.BlockSpec((1, G), lambda r, t: (0, 0)),
            pl.BlockSpec((1, hid), lambda r, t: (0, 0)),
            pl.BlockSpec((1, hid), lambda r, t: (0, 0)),
        ],
        out_specs=[
            pl.BlockSpec((1, _RB, 2 * hid), lambda r, t: (t, r, 0)),
            pl.BlockSpec((1, _RB, hid), lambda r, t: (t, r, 0)),
            pl.BlockSpec((_RB, hid), lambda r, t: (r, 0)),
            pl.BlockSpec((_RB, hid), lambda r, t: (r, 0)),
        ],
        out_shape=[
            jax.ShapeDtypeStruct((T, n, 2 * hid), jnp.float32),
            jax.ShapeDtypeStruct((T, n, hid), jnp.float32),
            jax.ShapeDtypeStruct((n, hid), jnp.float32),
            jax.ShapeDtypeStruct((n, hid), jnp.float32),
        ],
    )(x, sx, deg_parts, w0, w1, bc, bhz, bhh)


def _tc_gates(sh, deg_parts, h, xzr_t, wzr0, wzr1, bzr, hid):
    n = h.shape[0]
    grid = (n // _RB,)

    def body(sh_ref, dp_ref, h_ref, xzr_ref, w0_ref, w1_ref, b_ref,
             z_ref, hr_ref, hrs_ref):
        dinv = _dinv_from_parts(dp_ref[...])
        lh = (sh_ref[0, 0] + sh_ref[0, 1]) * (-dinv[:, None])
        hv = h_ref[...]
        pre = (xzr_ref[...]
               + jnp.dot(hv, w0_ref[...], preferred_element_type=jnp.float32)
               + jnp.dot(lh, w1_ref[...], preferred_element_type=jnp.float32)
               + b_ref[...])
        z = jax.nn.sigmoid(pre[:, :hid])
        r = jax.nn.sigmoid(pre[:, hid:])
        hr = hv * r
        z_ref[...] = z
        hr_ref[...] = hr
        hrs_ref[...] = hr * dinv[:, None]

    return pl.pallas_call(
        body,
        grid=grid,
        in_specs=[
            pl.BlockSpec((1, 2, _RB, hid), lambda r: (0, 0, r, 0)),
            pl.BlockSpec((_RB, 2), lambda r: (r, 0)),
            pl.BlockSpec((_RB, hid), lambda r: (r, 0)),
            pl.BlockSpec((_RB, 2 * hid), lambda r: (r, 0)),
            pl.BlockSpec((hid, 2 * hid), lambda r: (0, 0)),
            pl.BlockSpec((hid, 2 * hid), lambda r: (0, 0)),
            pl.BlockSpec((1, 2 * hid), lambda r: (0, 0)),
        ],
        out_specs=[
            pl.BlockSpec((_RB, hid), lambda r: (r, 0)),
            pl.BlockSpec((_RB, hid), lambda r: (r, 0)),
            pl.BlockSpec((_RB, hid), lambda r: (r, 0)),
        ],
        out_shape=[
            jax.ShapeDtypeStruct((n, hid), jnp.float32),
            jax.ShapeDtypeStruct((n, hid), jnp.float32),
            jax.ShapeDtypeStruct((n, hid), jnp.float32),
        ],
    )(sh, deg_parts, h, xzr_t, wzr0, wzr1, bzr)


def _tc_update(shr, deg_parts, h, z, hr, xh_t, whh0, whh1, bhh,
               hid, final, wl=None, bl=None, out_ch=None):
    n = h.shape[0]
    grid = (n // _RB,)

    def body(*refs):
        if final:
            (shr_ref, dp_ref, h_ref, z_ref, hr_ref, xh_ref, w0_ref, w1_ref,
             b_ref, wl_ref, bl_ref, hn_ref, hs_ref, out_ref) = refs
        else:
            (shr_ref, dp_ref, h_ref, z_ref, hr_ref, xh_ref, w0_ref, w1_ref,
             b_ref, hn_ref, hs_ref) = refs
        dinv = _dinv_from_parts(dp_ref[...])
        lhr = (shr_ref[0, 0] + shr_ref[0, 1]) * (-dinv[:, None])
        pre = (xh_ref[...]
               + jnp.dot(hr_ref[...], w0_ref[...], preferred_element_type=jnp.float32)
               + jnp.dot(lhr, w1_ref[...], preferred_element_type=jnp.float32)
               + b_ref[...])
        z = z_ref[...]
        hn = z * h_ref[...] + (1.0 - z) * jnp.tanh(pre)
        hn_ref[...] = hn
        hs_ref[...] = hn * dinv[:, None]
        if final:
            out_ref[...] = (jnp.dot(jnp.maximum(hn, 0.0), wl_ref[...],
                                    preferred_element_type=jnp.float32)
                            + bl_ref[...])

    in_specs = [
        pl.BlockSpec((1, 2, _RB, hid), lambda r: (0, 0, r, 0)),
        pl.BlockSpec((_RB, 2), lambda r: (r, 0)),
        pl.BlockSpec((_RB, hid), lambda r: (r, 0)),
        pl.BlockSpec((_RB, hid), lambda r: (r, 0)),
        pl.BlockSpec((_RB, hid), lambda r: (r, 0)),
        pl.BlockSpec((_RB, hid), lambda r: (r, 0)),
        pl.BlockSpec((hid, hid), lambda r: (0, 0)),
        pl.BlockSpec((hid, hid), lambda r: (0, 0)),
        pl.BlockSpec((1, hid), lambda r: (0, 0)),
    ]
    out_specs = [
        pl.BlockSpec((_RB, hid), lambda r: (r, 0)),
        pl.BlockSpec((_RB, hid), lambda r: (r, 0)),
    ]
    out_shape = [
        jax.ShapeDtypeStruct((n, hid), jnp.float32),
        jax.ShapeDtypeStruct((n, hid), jnp.float32),
    ]
    args = [shr, deg_parts, h, z, hr, xh_t, whh0, whh1, bhh]
    if final:
        in_specs += [pl.BlockSpec((hid, out_ch), lambda r: (0, 0)),
                     pl.BlockSpec((1, out_ch), lambda r: (0, 0))]
        out_specs.append(pl.BlockSpec((_RB, out_ch), lambda r: (r, 0)))
        out_shape.append(jax.ShapeDtypeStruct((n, out_ch), jnp.float32))
        args += [wl, bl]

    return pl.pallas_call(
        body, grid=grid, in_specs=in_specs, out_specs=out_specs,
        out_shape=out_shape,
    )(*args)


def kernel(x_seq, Wxz, bxz, Whz, bhz, Wxr, bxr, Whr, bhr, Wxh, bxh, Whh, bhh,
           Wl, bl, edge_index):
    if x_seq.ndim == 3:
        x_seq = x_seq[None]
    B, T, n, C = x_seq.shape
    hid = Whz.shape[1]
    out_ch = Wl.shape[1]
    n_edges = edge_index.shape[1]

    e_per_w = n_edges // (NC * NS)
    nwin = e_per_w // WIN
    src = edge_index[0].reshape(NC * NS, nwin, WIN)
    dst = edge_index[1].reshape(NC * NS, nwin, WIN)

    w0 = jnp.concatenate([Wxz[0], Wxr[0], Wxh[0]], axis=1)
    w1 = jnp.concatenate([Wxz[1], Wxr[1], Wxh[1]], axis=1)
    bc = jnp.concatenate([bxz, bxr, bxh])[None, :]
    wzr0 = jnp.concatenate([Whz[0], Whr[0]], axis=1)
    wzr1 = jnp.concatenate([Whz[1], Whr[1]], axis=1)
    bzr = jnp.concatenate([bhz, bhr])[None, :]
    whh0, whh1 = Whh[0], Whh[1]
    bhh2 = bhh[None, :]
    bl2 = bl[None, :]

    deg_k = _make_degree(n, n_edges)
    spmm_x = _make_spmm(n, n_edges, T, C)
    spmm_h = _make_spmm(n, n_edges, 1, hid)

    zeros_deg = jnp.zeros((-(-n // (NS * 128)) * 128,), jnp.float32)
    zeros_x = jnp.zeros((n // NS, C), jnp.float32)
    zeros_h = jnp.zeros((n // NS, hid), jnp.float32)

    deg_parts = deg_k(src, zeros_deg).T

    outs = []
    for b in range(B):
        x = x_seq[b]
        xs = _tc_prep(x, deg_parts)
        sx = spmm_x(*[xs[t] for t in range(T)], src, dst, zeros_x)
        xzr, xh, h, hs = _tc_xpre(x, sx, deg_parts, w0, w1, bc,
                                  bhz[None, :], bhh2, hid)
        for t in range(1, T):
            sh = spmm_h(hs, src, dst, zeros_h)
            z, hr, hrs = _tc_gates(sh, deg_parts, h, xzr[t], wzr0, wzr1,
                                   bzr, hid)
            shr = spmm_h(hrs, src, dst, zeros_h)
            final = t == T - 1
            if final:
                h, hs, out_b = _tc_update(shr, deg_parts, h, z, hr, xh[t],
                                          whh0, whh1, bhh2, hid, True,
                                          wl=Wl, bl=bl2, out_ch=out_ch)
            else:
                h, hs = _tc_update(shr, deg_parts, h, z, hr, xh[t],
                                   whh0, whh1, bhh2, hid, False)
        outs.append(out_b)
    return jnp.stack(outs, axis=0)


# speedup vs baseline: 1.0615x; 1.0615x over previous
import functools

import jax
import jax.numpy as jnp
from jax import lax
from jax.experimental import pallas as pl
from jax.experimental.pallas import tpu as pltpu
import jax.experimental.pallas.tpu_sc as plsc

NC = 2
NS = 16
WIN = 80


def _mesh():
    return plsc.VectorSubcoreMesh(core_axis_name="c", subcore_axis_name="s",
                                  num_cores=NC, num_subcores=NS)


@functools.lru_cache(maxsize=None)
def _make_degree(n_nodes, n_edges):
    e_per_w = n_edges // (NC * NS)
    nwin = e_per_w // WIN
    chunk = -(-n_nodes // (NS * 128)) * 128
    n_pad = NS * chunk

    @functools.partial(
        pl.kernel,
        mesh=_mesh(),
        out_type=jax.ShapeDtypeStruct((NC, n_pad), jnp.float32),
        scratch_types=[
            pltpu.VMEM((nwin, WIN), jnp.int32),
            pltpu.VMEM((WIN,), jnp.float32),
            pltpu.VMEM((chunk,), jnp.float32),
            pltpu.VMEM_SHARED((n_pad,), jnp.float32),
            pltpu.SemaphoreType.DMA,
        ],
    )
    def deg_kernel(src_hbm, zeros_hbm, out_hbm, idx_v, ones_v, z_v, acc, *sem_r):
        c = lax.axis_index("c")
        s = lax.axis_index("s")
        w = c * NS + s
        for i in range(WIN // 16):
            ones_v[pl.ds(16 * i, 16)] = jnp.ones((16,), jnp.float32)
        pltpu.sync_copy(zeros_hbm, z_v)
        pltpu.sync_copy(src_hbm.at[w], idx_v)
        pltpu.sync_copy(z_v, acc.at[pl.ds(s * chunk, chunk)])
        plsc.subcore_barrier()

        K = 4

        def body(j, carry):
            pltpu.async_copy(ones_v, acc.at[idx_v.at[j]], sem_r[0], add=True)
            @pl.when(j >= K - 1)
            def _():
                pltpu.make_async_copy(ones_v, acc.at[idx_v.at[j]],
                                      sem_r[0]).wait()
            return carry

        lax.fori_loop(0, nwin, body, 0)
        for _ in range(K - 1):
            pltpu.make_async_copy(ones_v, acc.at[idx_v.at[0]], sem_r[0]).wait()
        plsc.subcore_barrier()
        pltpu.sync_copy(acc.at[pl.ds(s * chunk, chunk)],
                        out_hbm.at[c, pl.ds(s * chunk, chunk)])

    return deg_kernel


@functools.lru_cache(maxsize=None)
def _make_spmm(n_nodes, n_edges, n_t, n_c):
    e_per_w = n_edges // (NC * NS)
    nwin = e_per_w // WIN
    nbuf = 3 if n_c >= 128 else 5
    rows_per_s = n_nodes // NS
    n_fori = (nwin - nbuf - (nbuf - 1)) // nbuf
    rest = list(range(nbuf + n_fori * nbuf, nwin))

    stage_tbl = False

    @functools.partial(
        pl.kernel,
        mesh=_mesh(),
        compiler_params=pltpu.CompilerParams(use_tc_tiling_on_sc=False),
        out_type=jax.ShapeDtypeStruct((n_t, NC, n_nodes, n_c), jnp.float32),
        scratch_types=(
            [pltpu.VMEM((nwin, WIN), jnp.int32)] * 2
            + [pltpu.VMEM((WIN, n_c), jnp.float32)] * nbuf
            + [pltpu.VMEM_SHARED((n_nodes, n_c), jnp.float32)]
            * (2 if stage_tbl else 1)
            + [pltpu.SemaphoreType.DMA] * (2 * nbuf)
        ),
    )
    def spmm_kernel(*args):
        tbls = args[:n_t]
        src_hbm, dst_hbm, zeros_hbm, out_hbm = args[n_t:n_t + 4]
        sidx, didx = args[n_t + 4:n_t + 6]
        bufs = args[n_t + 6:n_t + 6 + nbuf]
        n_sh = 2 if stage_tbl else 1
        acc = args[n_t + 6 + nbuf]
        tbl_sp = args[n_t + 7 + nbuf] if stage_tbl else None
        sem_g = args[n_t + 6 + nbuf + n_sh:n_t + 6 + nbuf + n_sh + nbuf]
        sem_s = args[n_t + 6 + nbuf + n_sh + nbuf:]
        c = lax.axis_index("c")
        s = lax.axis_index("s")
        w = c * NS + s
        rps = rows_per_s
        pltpu.sync_copy(src_hbm.at[w], sidx)
        pltpu.sync_copy(dst_hbm.at[w], didx)

        for t in range(n_t):
            if stage_tbl:
                pltpu.sync_copy(tbls[t].at[pl.ds(s * rps, rps), :],
                                tbl_sp.at[pl.ds(s * rps, rps), :])
                tbl = tbl_sp
            else:
                tbl = tbls[t]

            def g_issue(j, b):
                pltpu.async_copy(tbl.at[sidx.at[j]], bufs[b], sem_g[b])

            def g_wait(j, b):
                pltpu.make_async_copy(tbl.at[sidx.at[j]], bufs[b],
                                      sem_g[b]).wait()

            def s_issue(j, b):
                pltpu.async_copy(bufs[b], acc.at[didx.at[j]], sem_s[b],
                                 add=True)

            def s_wait(j, b):
                pltpu.make_async_copy(bufs[b], acc.at[didx.at[j]],
                                      sem_s[b]).wait()

            def step(j, b, full, first=False):
                g_wait(j, b)
                s_issue(j, b)
                if full:
                    bn = (b + nbuf - 1) % nbuf
                    if not first:
                        s_wait(j - 1, bn)
                    g_issue(j + nbuf - 1, bn)

            pltpu.sync_copy(
                zeros_hbm, acc.at[pl.ds(s * rows_per_s, rows_per_s), :])
            plsc.subcore_barrier()

            for b in range(nbuf - 1):
                g_issue(b, b)
            for j in range(nbuf):
                step(j, j % nbuf, full=True, first=(j == 0))

            def group(q, carry):
                for b in range(nbuf):
                    step(q * nbuf + b, b, full=True)
                return carry

            lax.fori_loop(1, 1 + n_fori, group, 0)
            for j in rest:
                step(j, j % nbuf, full=(j <= nwin - nbuf))
            for b in range(nbuf):
                s_wait(nwin - nbuf + b, (nwin - nbuf + b) % nbuf)
            plsc.subcore_barrier()
            pltpu.sync_copy(
                acc.at[pl.ds(s * rows_per_s, rows_per_s), :],
                out_hbm.at[t, c, pl.ds(s * rows_per_s, rows_per_s), :])
            plsc.subcore_barrier()

    return spmm_kernel


_RB = 2000


def _dinv_from_parts(parts):
    deg = parts[:, 0] + parts[:, 1]
    return jnp.where(deg > 0, lax.rsqrt(deg), 0.0)


def _tc_prep(x, deg_parts):
    T, n, C = x.shape
    grid = (T, n // _RB)

    def body(x_ref, dp_ref, xs_ref):
        dinv = _dinv_from_parts(dp_ref[...])
        xs_ref[0] = x_ref[0] * dinv[:, None]

    return pl.pallas_call(
        body,
        grid=grid,
        in_specs=[
            pl.BlockSpec((1, _RB, C), lambda t, r: (t, r, 0)),
            pl.BlockSpec((_RB, 2), lambda t, r: (r, 0)),
        ],
        out_specs=pl.BlockSpec((1, _RB, C), lambda t, r: (t, r, 0)),
        out_shape=jax.ShapeDtypeStruct((T, n, C), jnp.float32),
    )(x, deg_parts)


def _tc_xpre(x, sx, deg_parts, w0, w1, bc, bhz, bhh, hid):
    T, n, C = x.shape
    G = w0.shape[1]
    grid = (n // _RB, T)

    def body(x_ref, sx_ref, dp_ref, w0_ref, w1_ref, bc_ref, bhz_ref, bhh_ref,
             xzr_ref, xh_ref, h_ref, hs_ref):
        t = pl.program_id(1)
        dinv = _dinv_from_parts(dp_ref[...])
        lx = (sx_ref[0, 0] + sx_ref[0, 1]) * (-dinv[:, None])
        pre = (jnp.dot(x_ref[0], w0_ref[...], preferred_element_type=jnp.float32)
               + jnp.dot(lx, w1_ref[...], preferred_element_type=jnp.float32)
               + bc_ref[...])
        xzr_ref[0] = pre[:, : 2 * hid]
        xh_ref[0] = pre[:, 2 * hid:]

        @pl.when(t == 0)
        def _():
            z = jax.nn.sigmoid(pre[:, :hid] + bhz_ref[...])
            ht = jnp.tanh(pre[:, 2 * hid:] + bhh_ref[...])
            h = (1.0 - z) * ht
            h_ref[...] = h
            hs_ref[...] = h * dinv[:, None]

    return pl.pallas_call(
        body,
        grid=grid,
        in_specs=[
            pl.BlockSpec((1, _RB, C), lambda r, t: (t, r, 0)),
            pl.BlockSpec((1, 2, _RB, C), lambda r, t: (t, 0, r, 0)),
            pl.BlockSpec((_RB, 2), lambda r, t: (r, 0)),
            pl.BlockSpec((C, G), lambda r, t: (0, 0)),
            pl.BlockSpec((C, G), lambda r, t: (0, 0)),
            pl.BlockSpec((1, G), lambda r, t: (0, 0)),
            pl.BlockSpec((1, hid), lambda r, t: (0, 0)),
            pl.BlockSpec((1, hid), lambda r, t: (0, 0)),
        ],
        out_specs=[
            pl.BlockSpec((1, _RB, 2 * hid), lambda r, t: (t, r, 0)),
            pl.BlockSpec((1, _RB, hid), lambda r, t: (t, r, 0)),
            pl.BlockSpec((_RB, hid), lambda r, t: (r, 0)),
            pl.BlockSpec((_RB, hid), lambda r, t: (r, 0)),
        ],
        out_shape=[
            jax.ShapeDtypeStruct((T, n, 2 * hid), jnp.float32),
            jax.ShapeDtypeStruct((T, n, hid), jnp.float32),
            jax.ShapeDtypeStruct((n, hid), jnp.float32),
            jax.ShapeDtypeStruct((n, hid), jnp.float32),
        ],
    )(x, sx, deg_parts, w0, w1, bc, bhz, bhh)


def _tc_gates(sh, deg_parts, h, xzr_t, wzr0, wzr1, bzr, hid):
    n = h.shape[0]
    grid = (n // _RB,)

    def body(sh_ref, dp_ref, h_ref, xzr_ref, w0_ref, w1_ref, b_ref,
             z_ref, hr_ref, hrs_ref):
        dinv = _dinv_from_parts(dp_ref[...])
        lh = (sh_ref[0, 0] + sh_ref[0, 1]) * (-dinv[:, None])
        hv = h_ref[...]
        pre = (xzr_ref[...]
               + jnp.dot(hv, w0_ref[...], preferred_element_type=jnp.float32)
               + jnp.dot(lh, w1_ref[...], preferred_element_type=jnp.float32)
               + b_ref[...])
        z = jax.nn.sigmoid(pre[:, :hid])
        r = jax.nn.sigmoid(pre[:, hid:])
        hr = hv * r
        z_ref[...] = z
        hr_ref[...] = hr
        hrs_ref[...] = hr * dinv[:, None]

    return pl.pallas_call(
        body,
        grid=grid,
        in_specs=[
            pl.BlockSpec((1, 2, _RB, hid), lambda r: (0, 0, r, 0)),
            pl.BlockSpec((_RB, 2), lambda r: (r, 0)),
            pl.BlockSpec((_RB, hid), lambda r: (r, 0)),
            pl.BlockSpec((_RB, 2 * hid), lambda r: (r, 0)),
            pl.BlockSpec((hid, 2 * hid), lambda r: (0, 0)),
            pl.BlockSpec((hid, 2 * hid), lambda r: (0, 0)),
            pl.BlockSpec((1, 2 * hid), lambda r: (0, 0)),
        ],
        out_specs=[
            pl.BlockSpec((_RB, hid), lambda r: (r, 0)),
            pl.BlockSpec((_RB, hid), lambda r: (r, 0)),
            pl.BlockSpec((_RB, hid), lambda r: (r, 0)),
        ],
        out_shape=[
            jax.ShapeDtypeStruct((n, hid), jnp.float32),
            jax.ShapeDtypeStruct((n, hid), jnp.float32),
            jax.ShapeDtypeStruct((n, hid), jnp.float32),
        ],
    )(sh, deg_parts, h, xzr_t, wzr0, wzr1, bzr)


def _tc_update(shr, deg_parts, h, z, hr, xh_t, whh0, whh1, bhh,
               hid, final, wl=None, bl=None, out_ch=None):
    n = h.shape[0]
    grid = (n // _RB,)

    def body(*refs):
        if final:
            (shr_ref, dp_ref, h_ref, z_ref, hr_ref, xh_ref, w0_ref, w1_ref,
             b_ref, wl_ref, bl_ref, hn_ref, hs_ref, out_ref) = refs
        else:
            (shr_ref, dp_ref, h_ref, z_ref, hr_ref, xh_ref, w0_ref, w1_ref,
             b_ref, hn_ref, hs_ref) = refs
        dinv = _dinv_from_parts(dp_ref[...])
        lhr = (shr_ref[0, 0] + shr_ref[0, 1]) * (-dinv[:, None])
        pre = (xh_ref[...]
               + jnp.dot(hr_ref[...], w0_ref[...], preferred_element_type=jnp.float32)
               + jnp.dot(lhr, w1_ref[...], preferred_element_type=jnp.float32)
               + b_ref[...])
        z = z_ref[...]
        hn = z * h_ref[...] + (1.0 - z) * jnp.tanh(pre)
        hn_ref[...] = hn
        hs_ref[...] = hn * dinv[:, None]
        if final:
            out_ref[...] = (jnp.dot(jnp.maximum(hn, 0.0), wl_ref[...],
                                    preferred_element_type=jnp.float32)
                            + bl_ref[...])

    in_specs = [
        pl.BlockSpec((1, 2, _RB, hid), lambda r: (0, 0, r, 0)),
        pl.BlockSpec((_RB, 2), lambda r: (r, 0)),
        pl.BlockSpec((_RB, hid), lambda r: (r, 0)),
        pl.BlockSpec((_RB, hid), lambda r: (r, 0)),
        pl.BlockSpec((_RB, hid), lambda r: (r, 0)),
        pl.BlockSpec((_RB, hid), lambda r: (r, 0)),
        pl.BlockSpec((hid, hid), lambda r: (0, 0)),
        pl.BlockSpec((hid, hid), lambda r: (0, 0)),
        pl.BlockSpec((1, hid), lambda r: (0, 0)),
    ]
    out_specs = [
        pl.BlockSpec((_RB, hid), lambda r: (r, 0)),
        pl.BlockSpec((_RB, hid), lambda r: (r, 0)),
    ]
    out_shape = [
        jax.ShapeDtypeStruct((n, hid), jnp.float32),
        jax.ShapeDtypeStruct((n, hid), jnp.float32),
    ]
    args = [shr, deg_parts, h, z, hr, xh_t, whh0, whh1, bhh]
    if final:
        in_specs += [pl.BlockSpec((hid, out_ch), lambda r: (0, 0)),
                     pl.BlockSpec((1, out_ch), lambda r: (0, 0))]
        out_specs.append(pl.BlockSpec((_RB, out_ch), lambda r: (r, 0)))
        out_shape.append(jax.ShapeDtypeStruct((n, out_ch), jnp.float32))
        args += [wl, bl]

    return pl.pallas_call(
        body, grid=grid, in_specs=in_specs, out_specs=out_specs,
        out_shape=out_shape,
    )(*args)


def kernel(x_seq, Wxz, bxz, Whz, bhz, Wxr, bxr, Whr, bhr, Wxh, bxh, Whh, bhh,
           Wl, bl, edge_index):
    if x_seq.ndim == 3:
        x_seq = x_seq[None]
    B, T, n, C = x_seq.shape
    hid = Whz.shape[1]
    out_ch = Wl.shape[1]
    n_edges = edge_index.shape[1]

    e_per_w = n_edges // (NC * NS)
    nwin = e_per_w // WIN
    src = edge_index[0].reshape(NC * NS, nwin, WIN)
    dst = edge_index[1].reshape(NC * NS, nwin, WIN)

    w0 = jnp.concatenate([Wxz[0], Wxr[0], Wxh[0]], axis=1)
    w1 = jnp.concatenate([Wxz[1], Wxr[1], Wxh[1]], axis=1)
    bc = jnp.concatenate([bxz, bxr, bxh])[None, :]
    wzr0 = jnp.concatenate([Whz[0], Whr[0]], axis=1)
    wzr1 = jnp.concatenate([Whz[1], Whr[1]], axis=1)
    bzr = jnp.concatenate([bhz, bhr])[None, :]
    whh0, whh1 = Whh[0], Whh[1]
    bhh2 = bhh[None, :]
    bl2 = bl[None, :]

    deg_k = _make_degree(n, n_edges)
    spmm_x = _make_spmm(n, n_edges, T, C)
    spmm_h = _make_spmm(n, n_edges, 1, hid)

    zeros_deg = jnp.zeros((-(-n // (NS * 128)) * 128,), jnp.float32)
    zeros_x = jnp.zeros((n // NS, C), jnp.float32)
    zeros_h = jnp.zeros((n // NS, hid), jnp.float32)

    deg_parts = deg_k(src, zeros_deg).T

    outs = []
    for b in range(B):
        x = x_seq[b]
        xs = _tc_prep(x, deg_parts)
        sx = spmm_x(*[xs[t] for t in range(T)], src, dst, zeros_x)
        xzr, xh, h, hs = _tc_xpre(x, sx, deg_parts, w0, w1, bc,
                                  bhz[None, :], bhh2, hid)
        for t in range(1, T):
            sh = spmm_h(hs, src, dst, zeros_h)
            z, hr, hrs = _tc_gates(sh, deg_parts, h, xzr[t], wzr0, wzr1,
                                   bzr, hid)
            shr = spmm_h(hrs, src, dst, zeros_h)
            final = t == T - 1
            if final:
                h, hs, out_b = _tc_update(shr, deg_parts, h, z, hr, xh[t],
                                          whh0, whh1, bhh2, hid, True,
                                          wl=Wl, bl=bl2, out_ch=out_ch)
            else:
                h, hs = _tc_update(shr, deg_parts, h, z, hr, xh[t],
                                   whh0, whh1, bhh2, hid, False)
        outs.append(out_b)
    return jnp.stack(outs, axis=0)


# speedup vs baseline: 1.0903x; 1.0271x over previous
import functools

import jax
import jax.numpy as jnp
from jax import lax
from jax.experimental import pallas as pl
from jax.experimental.pallas import tpu as pltpu
import jax.experimental.pallas.tpu_sc as plsc

NC = 2
NS = 16
WIN = 80


def _mesh():
    return plsc.VectorSubcoreMesh(core_axis_name="c", subcore_axis_name="s",
                                  num_cores=NC, num_subcores=NS)


@functools.lru_cache(maxsize=None)
def _make_degree(n_nodes, n_edges):
    e_per_w = n_edges // (NC * NS)
    nwin = e_per_w // WIN
    chunk = -(-n_nodes // (NS * 128)) * 128
    n_pad = NS * chunk

    @functools.partial(
        pl.kernel,
        mesh=_mesh(),
        out_type=jax.ShapeDtypeStruct((NC, n_pad), jnp.float32),
        scratch_types=[
            pltpu.VMEM((nwin, WIN), jnp.int32),
            pltpu.VMEM((WIN,), jnp.float32),
            pltpu.VMEM((chunk,), jnp.float32),
            pltpu.VMEM_SHARED((n_pad,), jnp.float32),
            pltpu.SemaphoreType.DMA,
        ],
    )
    def deg_kernel(src_hbm, zeros_hbm, out_hbm, idx_v, ones_v, z_v, acc, *sem_r):
        c = lax.axis_index("c")
        s = lax.axis_index("s")
        w = c * NS + s
        for i in range(WIN // 16):
            ones_v[pl.ds(16 * i, 16)] = jnp.ones((16,), jnp.float32)
        pltpu.sync_copy(zeros_hbm, z_v)
        pltpu.sync_copy(src_hbm.at[w], idx_v)
        pltpu.sync_copy(z_v, acc.at[pl.ds(s * chunk, chunk)])
        plsc.subcore_barrier()

        K = 4

        def body(j, carry):
            pltpu.async_copy(ones_v, acc.at[idx_v.at[j]], sem_r[0], add=True)
            @pl.when(j >= K - 1)
            def _():
                pltpu.make_async_copy(ones_v, acc.at[idx_v.at[j]],
                                      sem_r[0]).wait()
            return carry

        lax.fori_loop(0, nwin, body, 0)
        for _ in range(K - 1):
            pltpu.make_async_copy(ones_v, acc.at[idx_v.at[0]], sem_r[0]).wait()
        plsc.subcore_barrier()
        pltpu.sync_copy(acc.at[pl.ds(s * chunk, chunk)],
                        out_hbm.at[c, pl.ds(s * chunk, chunk)])

    return deg_kernel


@functools.lru_cache(maxsize=None)
def _make_spmm(n_nodes, n_edges, n_t, n_c):
    e_per_w = n_edges // (NC * NS)
    nwin = e_per_w // WIN
    nbuf = 3 if n_c >= 128 else 5
    rows_per_s = n_nodes // NS
    n_fori = (nwin - nbuf - (nbuf - 1)) // nbuf
    rest = list(range(nbuf + n_fori * nbuf, nwin))

    stage_tbl = False

    @functools.partial(
        pl.kernel,
        mesh=_mesh(),
        compiler_params=pltpu.CompilerParams(use_tc_tiling_on_sc=False),
        out_type=jax.ShapeDtypeStruct((n_t, NC, n_nodes, n_c), jnp.float32),
        scratch_types=(
            [pltpu.VMEM((nwin, WIN), jnp.int32)] * 2
            + [pltpu.VMEM((WIN, n_c), jnp.float32)] * nbuf
            + [pltpu.VMEM_SHARED((n_nodes, n_c), jnp.float32)]
            * (2 if stage_tbl else 1)
            + [pltpu.SemaphoreType.DMA] * (2 * nbuf)
        ),
    )
    def spmm_kernel(*args):
        tbls = args[:n_t]
        src_hbm, dst_hbm, zeros_hbm, out_hbm = args[n_t:n_t + 4]
        sidx, didx = args[n_t + 4:n_t + 6]
        bufs = args[n_t + 6:n_t + 6 + nbuf]
        n_sh = 2 if stage_tbl else 1
        acc = args[n_t + 6 + nbuf]
        tbl_sp = args[n_t + 7 + nbuf] if stage_tbl else None
        sem_g = args[n_t + 6 + nbuf + n_sh:n_t + 6 + nbuf + n_sh + nbuf]
        sem_s = args[n_t + 6 + nbuf + n_sh + nbuf:]
        c = lax.axis_index("c")
        s = lax.axis_index("s")
        w = c * NS + s
        rps = rows_per_s
        pltpu.sync_copy(src_hbm.at[w], sidx)
        pltpu.sync_copy(dst_hbm.at[w], didx)

        for t in range(n_t):
            if stage_tbl:
                pltpu.sync_copy(tbls[t].at[pl.ds(s * rps, rps), :],
                                tbl_sp.at[pl.ds(s * rps, rps), :])
                tbl = tbl_sp
            else:
                tbl = tbls[t]

            def g_issue(j, b):
                pltpu.async_copy(tbl.at[sidx.at[j]], bufs[b], sem_g[b])

            def g_wait(j, b):
                pltpu.make_async_copy(tbl.at[sidx.at[j]], bufs[b],
                                      sem_g[b]).wait()

            def s_issue(j, b):
                pltpu.async_copy(bufs[b], acc.at[didx.at[j]], sem_s[b],
                                 add=True)

            def s_wait(j, b):
                pltpu.make_async_copy(bufs[b], acc.at[didx.at[j]],
                                      sem_s[b]).wait()

            def step(j, b, full, first=False):
                g_wait(j, b)
                s_issue(j, b)
                if full:
                    bn = (b + nbuf - 1) % nbuf
                    if not first:
                        s_wait(j - 1, bn)
                    g_issue(j + nbuf - 1, bn)

            pltpu.sync_copy(
                zeros_hbm, acc.at[pl.ds(s * rows_per_s, rows_per_s), :])
            plsc.subcore_barrier()

            for b in range(nbuf - 1):
                g_issue(b, b)
            for j in range(nbuf):
                step(j, j % nbuf, full=True, first=(j == 0))

            def group(q, carry):
                for b in range(nbuf):
                    step(q * nbuf + b, b, full=True)
                return carry

            lax.fori_loop(1, 1 + n_fori, group, 0)
            for j in rest:
                step(j, j % nbuf, full=(j <= nwin - nbuf))
            for b in range(nbuf):
                s_wait(nwin - nbuf + b, (nwin - nbuf + b) % nbuf)
            plsc.subcore_barrier()
            pltpu.sync_copy(
                acc.at[pl.ds(s * rows_per_s, rows_per_s), :],
                out_hbm.at[t, c, pl.ds(s * rows_per_s, rows_per_s), :])
            plsc.subcore_barrier()

    return spmm_kernel


_RB = 2000


def _dinv_from_parts(parts):
    deg = parts[:, 0] + parts[:, 1]
    return jnp.where(deg > 0, lax.rsqrt(deg), 0.0)


def _tc_prep(x, deg_parts):
    T, n, C = x.shape
    grid = (T, n // _RB)

    def body(x_ref, dp_ref, xs_ref):
        dinv = _dinv_from_parts(dp_ref[...])
        xs_ref[0] = x_ref[0] * dinv[:, None]

    return pl.pallas_call(
        body,
        grid=grid,
        in_specs=[
            pl.BlockSpec((1, _RB, C), lambda t, r: (t, r, 0)),
            pl.BlockSpec((_RB, 2), lambda t, r: (r, 0)),
        ],
        out_specs=pl.BlockSpec((1, _RB, C), lambda t, r: (t, r, 0)),
        out_shape=jax.ShapeDtypeStruct((T, n, C), jnp.float32),
    )(x, deg_parts)


@functools.lru_cache(maxsize=None)
def _tc_t0init_builder(T, n, C, hid):
    grid = (n // _RB,)

    def body(x_ref, sx_ref, dp_ref, w0_ref, w1_ref, bc_ref, bhz_ref, bhh_ref,
             h_ref, hs_ref):
        dinv = _dinv_from_parts(dp_ref[...])
        lx = (sx_ref[0, 0] + sx_ref[0, 1]) * (-dinv[:, None])
        pre = (jnp.dot(x_ref[0], w0_ref[...], preferred_element_type=jnp.float32)
               + jnp.dot(lx, w1_ref[...], preferred_element_type=jnp.float32)
               + bc_ref[...])
        z = jax.nn.sigmoid(pre[:, :hid] + bhz_ref[...])
        ht = jnp.tanh(pre[:, 2 * hid:] + bhh_ref[...])
        h = (1.0 - z) * ht
        h_ref[...] = h
        hs_ref[...] = h * dinv[:, None]

    G = 3 * hid
    return pl.pallas_call(
        body,
        grid=grid,
        in_specs=[
            pl.BlockSpec((1, _RB, C), lambda r: (0, r, 0)),
            pl.BlockSpec((1, 2, _RB, C), lambda r: (0, 0, r, 0)),
            pl.BlockSpec((_RB, 2), lambda r: (r, 0)),
            pl.BlockSpec((C, G), lambda r: (0, 0)),
            pl.BlockSpec((C, G), lambda r: (0, 0)),
            pl.BlockSpec((1, G), lambda r: (0, 0)),
            pl.BlockSpec((1, hid), lambda r: (0, 0)),
            pl.BlockSpec((1, hid), lambda r: (0, 0)),
        ],
        out_specs=[
            pl.BlockSpec((_RB, hid), lambda r: (r, 0)),
            pl.BlockSpec((_RB, hid), lambda r: (r, 0)),
        ],
        out_shape=[
            jax.ShapeDtypeStruct((n, hid), jnp.float32),
            jax.ShapeDtypeStruct((n, hid), jnp.float32),
        ],
    )


def _tc_t0init(x, sx, deg_parts, w0, w1, bc, bhz, bhh, hid):
    T, n, C = x.shape
    return _tc_t0init_builder(T, n, C, hid)(
        x, sx, deg_parts, w0, w1, bc, bhz, bhh)


@functools.lru_cache(maxsize=None)
def _tc_gates_builder(t, T, n, C, hid):
    grid = (n // _RB,)
    G = 3 * hid

    def body(x_ref, sx_ref, sh_ref, dp_ref, h_ref, w0_ref, w1_ref, bc_ref,
             wzr0_ref, wzr1_ref, bzr_ref, z_ref, hr_ref, hrs_ref, xh_ref):
        dinv = _dinv_from_parts(dp_ref[...])
        lx = (sx_ref[0, 0] + sx_ref[0, 1]) * (-dinv[:, None])
        prex = (jnp.dot(x_ref[0], w0_ref[...], preferred_element_type=jnp.float32)
                + jnp.dot(lx, w1_ref[...], preferred_element_type=jnp.float32)
                + bc_ref[...])
        xh_ref[...] = prex[:, 2 * hid:]
        lh = (sh_ref[0, 0] + sh_ref[0, 1]) * (-dinv[:, None])
        hv = h_ref[...]
        pre = (prex[:, :2 * hid]
               + jnp.dot(hv, wzr0_ref[...], preferred_element_type=jnp.float32)
               + jnp.dot(lh, wzr1_ref[...], preferred_element_type=jnp.float32)
               + bzr_ref[...])
        z = jax.nn.sigmoid(pre[:, :hid])
        r = jax.nn.sigmoid(pre[:, hid:])
        hr = hv * r
        z_ref[...] = z
        hr_ref[...] = hr
        hrs_ref[...] = hr * dinv[:, None]

    return pl.pallas_call(
        body,
        grid=grid,
        in_specs=[
            pl.BlockSpec((1, _RB, C), lambda r: (t, r, 0)),
            pl.BlockSpec((1, 2, _RB, C), lambda r: (t, 0, r, 0)),
            pl.BlockSpec((1, 2, _RB, hid), lambda r: (0, 0, r, 0)),
            pl.BlockSpec((_RB, 2), lambda r: (r, 0)),
            pl.BlockSpec((_RB, hid), lambda r: (r, 0)),
            pl.BlockSpec((C, G), lambda r: (0, 0)),
            pl.BlockSpec((C, G), lambda r: (0, 0)),
            pl.BlockSpec((1, G), lambda r: (0, 0)),
            pl.BlockSpec((hid, 2 * hid), lambda r: (0, 0)),
            pl.BlockSpec((hid, 2 * hid), lambda r: (0, 0)),
            pl.BlockSpec((1, 2 * hid), lambda r: (0, 0)),
        ],
        out_specs=[
            pl.BlockSpec((_RB, hid), lambda r: (r, 0)),
            pl.BlockSpec((_RB, hid), lambda r: (r, 0)),
            pl.BlockSpec((_RB, hid), lambda r: (r, 0)),
            pl.BlockSpec((_RB, hid), lambda r: (r, 0)),
        ],
        out_shape=[
            jax.ShapeDtypeStruct((n, hid), jnp.float32),
            jax.ShapeDtypeStruct((n, hid), jnp.float32),
            jax.ShapeDtypeStruct((n, hid), jnp.float32),
            jax.ShapeDtypeStruct((n, hid), jnp.float32),
        ],
    )


def _tc_gates(t, x, sx, sh, deg_parts, h, w0, w1, bc, wzr0, wzr1, bzr, hid):
    T, n, C = x.shape
    return _tc_gates_builder(t, T, n, C, hid)(
        x, sx, sh, deg_parts, h, w0, w1, bc, wzr0, wzr1, bzr)


def _tc_update(shr, deg_parts, h, z, hr, xh_t, whh0, whh1, bhh,
               hid, final, wl=None, bl=None, out_ch=None):
    n = h.shape[0]
    grid = (n // _RB,)

    def body(*refs):
        if final:
            (shr_ref, dp_ref, h_ref, z_ref, hr_ref, xh_ref, w0_ref, w1_ref,
             b_ref, wl_ref, bl_ref, hn_ref, hs_ref, out_ref) = refs
        else:
            (shr_ref, dp_ref, h_ref, z_ref, hr_ref, xh_ref, w0_ref, w1_ref,
             b_ref, hn_ref, hs_ref) = refs
        dinv = _dinv_from_parts(dp_ref[...])
        lhr = (shr_ref[0, 0] + shr_ref[0, 1]) * (-dinv[:, None])
        pre = (xh_ref[...]
               + jnp.dot(hr_ref[...], w0_ref[...], preferred_element_type=jnp.float32)
               + jnp.dot(lhr, w1_ref[...], preferred_element_type=jnp.float32)
               + b_ref[...])
        z = z_ref[...]
        hn = z * h_ref[...] + (1.0 - z) * jnp.tanh(pre)
        hn_ref[...] = hn
        hs_ref[...] = hn * dinv[:, None]
        if final:
            out_ref[...] = (jnp.dot(jnp.maximum(hn, 0.0), wl_ref[...],
                                    preferred_element_type=jnp.float32)
                            + bl_ref[...])

    in_specs = [
        pl.BlockSpec((1, 2, _RB, hid), lambda r: (0, 0, r, 0)),
        pl.BlockSpec((_RB, 2), lambda r: (r, 0)),
        pl.BlockSpec((_RB, hid), lambda r: (r, 0)),
        pl.BlockSpec((_RB, hid), lambda r: (r, 0)),
        pl.BlockSpec((_RB, hid), lambda r: (r, 0)),
        pl.BlockSpec((_RB, hid), lambda r: (r, 0)),
        pl.BlockSpec((hid, hid), lambda r: (0, 0)),
        pl.BlockSpec((hid, hid), lambda r: (0, 0)),
        pl.BlockSpec((1, hid), lambda r: (0, 0)),
    ]
    out_specs = [
        pl.BlockSpec((_RB, hid), lambda r: (r, 0)),
        pl.BlockSpec((_RB, hid), lambda r: (r, 0)),
    ]
    out_shape = [
        jax.ShapeDtypeStruct((n, hid), jnp.float32),
        jax.ShapeDtypeStruct((n, hid), jnp.float32),
    ]
    args = [shr, deg_parts, h, z, hr, xh_t, whh0, whh1, bhh]
    if final:
        in_specs += [pl.BlockSpec((hid, out_ch), lambda r: (0, 0)),
                     pl.BlockSpec((1, out_ch), lambda r: (0, 0))]
        out_specs.append(pl.BlockSpec((_RB, out_ch), lambda r: (r, 0)))
        out_shape.append(jax.ShapeDtypeStruct((n, out_ch), jnp.float32))
        args += [wl, bl]

    return pl.pallas_call(
        body, grid=grid, in_specs=in_specs, out_specs=out_specs,
        out_shape=out_shape,
    )(*args)


def kernel(x_seq, Wxz, bxz, Whz, bhz, Wxr, bxr, Whr, bhr, Wxh, bxh, Whh, bhh,
           Wl, bl, edge_index):
    if x_seq.ndim == 3:
        x_seq = x_seq[None]
    B, T, n, C = x_seq.shape
    hid = Whz.shape[1]
    out_ch = Wl.shape[1]
    n_edges = edge_index.shape[1]

    e_per_w = n_edges // (NC * NS)
    nwin = e_per_w // WIN
    src = edge_index[0].reshape(NC * NS, nwin, WIN)
    dst = edge_index[1].reshape(NC * NS, nwin, WIN)

    w0 = jnp.concatenate([Wxz[0], Wxr[0], Wxh[0]], axis=1)
    w1 = jnp.concatenate([Wxz[1], Wxr[1], Wxh[1]], axis=1)
    bc = jnp.concatenate([bxz, bxr, bxh])[None, :]
    wzr0 = jnp.concatenate([Whz[0], Whr[0]], axis=1)
    wzr1 = jnp.concatenate([Whz[1], Whr[1]], axis=1)
    bzr = jnp.concatenate([bhz, bhr])[None, :]
    whh0, whh1 = Whh[0], Whh[1]
    bhh2 = bhh[None, :]
    bl2 = bl[None, :]

    deg_k = _make_degree(n, n_edges)
    spmm_x = _make_spmm(n, n_edges, T, C)
    spmm_h = _make_spmm(n, n_edges, 1, hid)

    zeros_deg = jnp.zeros((-(-n // (NS * 128)) * 128,), jnp.float32)
    zeros_x = jnp.zeros((n // NS, C), jnp.float32)
    zeros_h = jnp.zeros((n // NS, hid), jnp.float32)

    deg_parts = deg_k(src, zeros_deg).T

    outs = []
    for b in range(B):
        x = x_seq[b]
        xs = _tc_prep(x, deg_parts)
        sx = spmm_x(*[xs[t] for t in range(T)], src, dst, zeros_x)
        h, hs = _tc_t0init(x, sx, deg_parts, w0, w1, bc,
                           bhz[None, :], bhh2, hid)
        for t in range(1, T):
            sh = spmm_h(hs, src, dst, zeros_h)
            z, hr, hrs, xhp = _tc_gates(t, x, sx, sh, deg_parts, h,
                                        w0, w1, bc, wzr0, wzr1, bzr, hid)
            shr = spmm_h(hrs, src, dst, zeros_h)
            final = t == T - 1
            if final:
                h, hs, out_b = _tc_update(shr, deg_parts, h, z, hr, xhp,
                                          whh0, whh1, bhh2, hid, True,
                                          wl=Wl, bl=bl2, out_ch=out_ch)
            else:
                h, hs = _tc_update(shr, deg_parts, h, z, hr, xhp,
                                   whh0, whh1, bhh2, hid, False)
        outs.append(out_b)
    return jnp.stack(outs, axis=0)


# speedup vs baseline: 1.1006x; 1.0095x over previous
import functools

import jax
import jax.numpy as jnp
from jax import lax
from jax.experimental import pallas as pl
from jax.experimental.pallas import tpu as pltpu
import jax.experimental.pallas.tpu_sc as plsc

NC = 2
NS = 16
WIN = 80


def _mesh():
    return plsc.VectorSubcoreMesh(core_axis_name="c", subcore_axis_name="s",
                                  num_cores=NC, num_subcores=NS)


@functools.lru_cache(maxsize=None)
def _make_degree(n_nodes, n_edges):
    e_per_w = n_edges // (NC * NS)
    nwin = e_per_w // WIN
    chunk = -(-n_nodes // (NS * 128)) * 128
    n_pad = NS * chunk

    @functools.partial(
        pl.kernel,
        mesh=_mesh(),
        out_type=jax.ShapeDtypeStruct((NC, n_pad), jnp.float32),
        scratch_types=[
            pltpu.VMEM((nwin, WIN), jnp.int32),
            pltpu.VMEM((WIN,), jnp.float32),
            pltpu.VMEM((chunk,), jnp.float32),
            pltpu.VMEM_SHARED((n_pad,), jnp.float32),
            pltpu.SemaphoreType.DMA,
        ],
    )
    def deg_kernel(src_hbm, zeros_hbm, out_hbm, idx_v, ones_v, z_v, acc, *sem_r):
        c = lax.axis_index("c")
        s = lax.axis_index("s")
        w = c * NS + s
        for i in range(WIN // 16):
            ones_v[pl.ds(16 * i, 16)] = jnp.ones((16,), jnp.float32)
        pltpu.sync_copy(zeros_hbm, z_v)
        pltpu.sync_copy(src_hbm.at[w], idx_v)
        pltpu.sync_copy(z_v, acc.at[pl.ds(s * chunk, chunk)])
        plsc.subcore_barrier()

        K = 4

        def body(j, carry):
            pltpu.async_copy(ones_v, acc.at[idx_v.at[j]], sem_r[0], add=True)
            @pl.when(j >= K - 1)
            def _():
                pltpu.make_async_copy(ones_v, acc.at[idx_v.at[j]],
                                      sem_r[0]).wait()
            return carry

        lax.fori_loop(0, nwin, body, 0)
        for _ in range(K - 1):
            pltpu.make_async_copy(ones_v, acc.at[idx_v.at[0]], sem_r[0]).wait()
        plsc.subcore_barrier()
        pltpu.sync_copy(acc.at[pl.ds(s * chunk, chunk)],
                        out_hbm.at[c, pl.ds(s * chunk, chunk)])

    return deg_kernel


@functools.lru_cache(maxsize=None)
def _make_spmm(n_nodes, n_edges, n_t, n_c):
    e_per_w = n_edges // (NC * NS)
    nwin = e_per_w // WIN
    nbuf = 3 if n_c >= 128 else 7
    rows_per_s = n_nodes // NS
    n_fori = (nwin - nbuf - (nbuf - 1)) // nbuf
    rest = list(range(nbuf + n_fori * nbuf, nwin))

    stage_tbl = False

    @functools.partial(
        pl.kernel,
        mesh=_mesh(),
        compiler_params=pltpu.CompilerParams(use_tc_tiling_on_sc=False),
        out_type=jax.ShapeDtypeStruct((n_t, NC, n_nodes, n_c), jnp.float32),
        scratch_types=(
            [pltpu.VMEM((nwin, WIN), jnp.int32)] * 2
            + [pltpu.VMEM((WIN, n_c), jnp.float32)] * nbuf
            + [pltpu.VMEM_SHARED((n_nodes, n_c), jnp.float32)]
            * (2 if stage_tbl else 1)
            + [pltpu.SemaphoreType.DMA] * (2 * nbuf)
        ),
    )
    def spmm_kernel(*args):
        tbls = args[:n_t]
        src_hbm, dst_hbm, zeros_hbm, out_hbm = args[n_t:n_t + 4]
        sidx, didx = args[n_t + 4:n_t + 6]
        bufs = args[n_t + 6:n_t + 6 + nbuf]
        n_sh = 2 if stage_tbl else 1
        acc = args[n_t + 6 + nbuf]
        tbl_sp = args[n_t + 7 + nbuf] if stage_tbl else None
        sem_g = args[n_t + 6 + nbuf + n_sh:n_t + 6 + nbuf + n_sh + nbuf]
        sem_s = args[n_t + 6 + nbuf + n_sh + nbuf:]
        c = lax.axis_index("c")
        s = lax.axis_index("s")
        w = c * NS + s
        rps = rows_per_s
        pltpu.sync_copy(src_hbm.at[w], sidx)
        pltpu.sync_copy(dst_hbm.at[w], didx)

        for t in range(n_t):
            if stage_tbl:
                pltpu.sync_copy(tbls[t].at[pl.ds(s * rps, rps), :],
                                tbl_sp.at[pl.ds(s * rps, rps), :])
                tbl = tbl_sp
            else:
                tbl = tbls[t]

            def g_issue(j, b):
                pltpu.async_copy(tbl.at[sidx.at[j]], bufs[b], sem_g[b])

            def g_wait(j, b):
                pltpu.make_async_copy(tbl.at[sidx.at[j]], bufs[b],
                                      sem_g[b]).wait()

            def s_issue(j, b):
                pltpu.async_copy(bufs[b], acc.at[didx.at[j]], sem_s[b],
                                 add=True)

            def s_wait(j, b):
                pltpu.make_async_copy(bufs[b], acc.at[didx.at[j]],
                                      sem_s[b]).wait()

            def step(j, b, full, first=False):
                g_wait(j, b)
                s_issue(j, b)
                if full:
                    bn = (b + nbuf - 1) % nbuf
                    if not first:
                        s_wait(j - 1, bn)
                    g_issue(j + nbuf - 1, bn)

            pltpu.sync_copy(
                zeros_hbm, acc.at[pl.ds(s * rows_per_s, rows_per_s), :])
            plsc.subcore_barrier()

            for b in range(nbuf - 1):
                g_issue(b, b)
            for j in range(nbuf):
                step(j, j % nbuf, full=True, first=(j == 0))

            def group(q, carry):
                for b in range(nbuf):
                    step(q * nbuf + b, b, full=True)
                return carry

            lax.fori_loop(1, 1 + n_fori, group, 0)
            for j in rest:
                step(j, j % nbuf, full=(j <= nwin - nbuf))
            for b in range(nbuf):
                s_wait(nwin - nbuf + b, (nwin - nbuf + b) % nbuf)
            plsc.subcore_barrier()
            pltpu.sync_copy(
                acc.at[pl.ds(s * rows_per_s, rows_per_s), :],
                out_hbm.at[t, c, pl.ds(s * rows_per_s, rows_per_s), :])
            plsc.subcore_barrier()

    return spmm_kernel


_RB = 2000


def _dinv_from_parts(parts):
    deg = parts[:, 0] + parts[:, 1]
    return jnp.where(deg > 0, lax.rsqrt(deg), 0.0)


def _tc_prep(x, deg_parts):
    T, n, C = x.shape
    grid = (T, n // _RB)

    def body(x_ref, dp_ref, xs_ref):
        dinv = _dinv_from_parts(dp_ref[...])
        xs_ref[0] = x_ref[0] * dinv[:, None]

    return pl.pallas_call(
        body,
        grid=grid,
        in_specs=[
            pl.BlockSpec((1, _RB, C), lambda t, r: (t, r, 0)),
            pl.BlockSpec((_RB, 2), lambda t, r: (r, 0)),
        ],
        out_specs=pl.BlockSpec((1, _RB, C), lambda t, r: (t, r, 0)),
        out_shape=jax.ShapeDtypeStruct((T, n, C), jnp.float32),
    )(x, deg_parts)


@functools.lru_cache(maxsize=None)
def _tc_t0init_builder(T, n, C, hid):
    grid = (n // _RB,)

    def body(x_ref, sx_ref, dp_ref, w0_ref, w1_ref, bc_ref, bhz_ref, bhh_ref,
             h_ref, hs_ref):
        dinv = _dinv_from_parts(dp_ref[...])
        lx = (sx_ref[0, 0] + sx_ref[0, 1]) * (-dinv[:, None])
        pre = (jnp.dot(x_ref[0], w0_ref[...], preferred_element_type=jnp.float32)
               + jnp.dot(lx, w1_ref[...], preferred_element_type=jnp.float32)
               + bc_ref[...])
        z = jax.nn.sigmoid(pre[:, :hid] + bhz_ref[...])
        ht = jnp.tanh(pre[:, 2 * hid:] + bhh_ref[...])
        h = (1.0 - z) * ht
        h_ref[...] = h
        hs_ref[...] = h * dinv[:, None]

    G = 3 * hid
    return pl.pallas_call(
        body,
        grid=grid,
        in_specs=[
            pl.BlockSpec((1, _RB, C), lambda r: (0, r, 0)),
            pl.BlockSpec((1, 2, _RB, C), lambda r: (0, 0, r, 0)),
            pl.BlockSpec((_RB, 2), lambda r: (r, 0)),
            pl.BlockSpec((C, G), lambda r: (0, 0)),
            pl.BlockSpec((C, G), lambda r: (0, 0)),
            pl.BlockSpec((1, G), lambda r: (0, 0)),
            pl.BlockSpec((1, hid), lambda r: (0, 0)),
            pl.BlockSpec((1, hid), lambda r: (0, 0)),
        ],
        out_specs=[
            pl.BlockSpec((_RB, hid), lambda r: (r, 0)),
            pl.BlockSpec((_RB, hid), lambda r: (r, 0)),
        ],
        out_shape=[
            jax.ShapeDtypeStruct((n, hid), jnp.float32),
            jax.ShapeDtypeStruct((n, hid), jnp.float32),
        ],
    )


def _tc_t0init(x, sx, deg_parts, w0, w1, bc, bhz, bhh, hid):
    T, n, C = x.shape
    return _tc_t0init_builder(T, n, C, hid)(
        x, sx, deg_parts, w0, w1, bc, bhz, bhh)


@functools.lru_cache(maxsize=None)
def _tc_gates_builder(t, T, n, C, hid):
    grid = (n // _RB,)
    G = 3 * hid

    def body(x_ref, sx_ref, sh_ref, dp_ref, h_ref, w0_ref, w1_ref, bc_ref,
             wzr0_ref, wzr1_ref, bzr_ref, z_ref, hr_ref, hrs_ref, xh_ref):
        dinv = _dinv_from_parts(dp_ref[...])
        lx = (sx_ref[0, 0] + sx_ref[0, 1]) * (-dinv[:, None])
        prex = (jnp.dot(x_ref[0], w0_ref[...], preferred_element_type=jnp.float32)
                + jnp.dot(lx, w1_ref[...], preferred_element_type=jnp.float32)
                + bc_ref[...])
        xh_ref[...] = prex[:, 2 * hid:]
        lh = (sh_ref[0, 0] + sh_ref[0, 1]) * (-dinv[:, None])
        hv = h_ref[...]
        pre = (prex[:, :2 * hid]
               + jnp.dot(hv, wzr0_ref[...], preferred_element_type=jnp.float32)
               + jnp.dot(lh, wzr1_ref[...], preferred_element_type=jnp.float32)
               + bzr_ref[...])
        z = jax.nn.sigmoid(pre[:, :hid])
        r = jax.nn.sigmoid(pre[:, hid:])
        hr = hv * r
        z_ref[...] = z
        hr_ref[...] = hr
        hrs_ref[...] = hr * dinv[:, None]

    return pl.pallas_call(
        body,
        grid=grid,
        in_specs=[
            pl.BlockSpec((1, _RB, C), lambda r: (t, r, 0)),
            pl.BlockSpec((1, 2, _RB, C), lambda r: (t, 0, r, 0)),
            pl.BlockSpec((1, 2, _RB, hid), lambda r: (0, 0, r, 0)),
            pl.BlockSpec((_RB, 2), lambda r: (r, 0)),
            pl.BlockSpec((_RB, hid), lambda r: (r, 0)),
            pl.BlockSpec((C, G), lambda r: (0, 0)),
            pl.BlockSpec((C, G), lambda r: (0, 0)),
            pl.BlockSpec((1, G), lambda r: (0, 0)),
            pl.BlockSpec((hid, 2 * hid), lambda r: (0, 0)),
            pl.BlockSpec((hid, 2 * hid), lambda r: (0, 0)),
            pl.BlockSpec((1, 2 * hid), lambda r: (0, 0)),
        ],
        out_specs=[
            pl.BlockSpec((_RB, hid), lambda r: (r, 0)),
            pl.BlockSpec((_RB, hid), lambda r: (r, 0)),
            pl.BlockSpec((_RB, hid), lambda r: (r, 0)),
            pl.BlockSpec((_RB, hid), lambda r: (r, 0)),
        ],
        out_shape=[
            jax.ShapeDtypeStruct((n, hid), jnp.float32),
            jax.ShapeDtypeStruct((n, hid), jnp.float32),
            jax.ShapeDtypeStruct((n, hid), jnp.float32),
            jax.ShapeDtypeStruct((n, hid), jnp.float32),
        ],
    )


def _tc_gates(t, x, sx, sh, deg_parts, h, w0, w1, bc, wzr0, wzr1, bzr, hid):
    T, n, C = x.shape
    return _tc_gates_builder(t, T, n, C, hid)(
        x, sx, sh, deg_parts, h, w0, w1, bc, wzr0, wzr1, bzr)


def _tc_update(shr, deg_parts, h, z, hr, xh_t, whh0, whh1, bhh,
               hid, final, wl=None, bl=None, out_ch=None):
    n = h.shape[0]
    grid = (n // _RB,)

    def body(*refs):
        if final:
            (shr_ref, dp_ref, h_ref, z_ref, hr_ref, xh_ref, w0_ref, w1_ref,
             b_ref, wl_ref, bl_ref, hn_ref, hs_ref, out_ref) = refs
        else:
            (shr_ref, dp_ref, h_ref, z_ref, hr_ref, xh_ref, w0_ref, w1_ref,
             b_ref, hn_ref, hs_ref) = refs
        dinv = _dinv_from_parts(dp_ref[...])
        lhr = (shr_ref[0, 0] + shr_ref[0, 1]) * (-dinv[:, None])
        pre = (xh_ref[...]
               + jnp.dot(hr_ref[...], w0_ref[...], preferred_element_type=jnp.float32)
               + jnp.dot(lhr, w1_ref[...], preferred_element_type=jnp.float32)
               + b_ref[...])
        z = z_ref[...]
        hn = z * h_ref[...] + (1.0 - z) * jnp.tanh(pre)
        hn_ref[...] = hn
        hs_ref[...] = hn * dinv[:, None]
        if final:
            out_ref[...] = (jnp.dot(jnp.maximum(hn, 0.0), wl_ref[...],
                                    preferred_element_type=jnp.float32)
                            + bl_ref[...])

    in_specs = [
        pl.BlockSpec((1, 2, _RB, hid), lambda r: (0, 0, r, 0)),
        pl.BlockSpec((_RB, 2), lambda r: (r, 0)),
        pl.BlockSpec((_RB, hid), lambda r: (r, 0)),
        pl.BlockSpec((_RB, hid), lambda r: (r, 0)),
        pl.BlockSpec((_RB, hid), lambda r: (r, 0)),
        pl.BlockSpec((_RB, hid), lambda r: (r, 0)),
        pl.BlockSpec((hid, hid), lambda r: (0, 0)),
        pl.BlockSpec((hid, hid), lambda r: (0, 0)),
        pl.BlockSpec((1, hid), lambda r: (0, 0)),
    ]
    out_specs = [
        pl.BlockSpec((_RB, hid), lambda r: (r, 0)),
        pl.BlockSpec((_RB, hid), lambda r: (r, 0)),
    ]
    out_shape = [
        jax.ShapeDtypeStruct((n, hid), jnp.float32),
        jax.ShapeDtypeStruct((n, hid), jnp.float32),
    ]
    args = [shr, deg_parts, h, z, hr, xh_t, whh0, whh1, bhh]
    if final:
        in_specs += [pl.BlockSpec((hid, out_ch), lambda r: (0, 0)),
                     pl.BlockSpec((1, out_ch), lambda r: (0, 0))]
        out_specs.append(pl.BlockSpec((_RB, out_ch), lambda r: (r, 0)))
        out_shape.append(jax.ShapeDtypeStruct((n, out_ch), jnp.float32))
        args += [wl, bl]

    return pl.pallas_call(
        body, grid=grid, in_specs=in_specs, out_specs=out_specs,
        out_shape=out_shape,
    )(*args)


def kernel(x_seq, Wxz, bxz, Whz, bhz, Wxr, bxr, Whr, bhr, Wxh, bxh, Whh, bhh,
           Wl, bl, edge_index):
    if x_seq.ndim == 3:
        x_seq = x_seq[None]
    B, T, n, C = x_seq.shape
    hid = Whz.shape[1]
    out_ch = Wl.shape[1]
    n_edges = edge_index.shape[1]

    e_per_w = n_edges // (NC * NS)
    nwin = e_per_w // WIN
    src = edge_index[0].reshape(NC * NS, nwin, WIN)
    dst = edge_index[1].reshape(NC * NS, nwin, WIN)

    w0 = jnp.concatenate([Wxz[0], Wxr[0], Wxh[0]], axis=1)
    w1 = jnp.concatenate([Wxz[1], Wxr[1], Wxh[1]], axis=1)
    bc = jnp.concatenate([bxz, bxr, bxh])[None, :]
    wzr0 = jnp.concatenate([Whz[0], Whr[0]], axis=1)
    wzr1 = jnp.concatenate([Whz[1], Whr[1]], axis=1)
    bzr = jnp.concatenate([bhz, bhr])[None, :]
    whh0, whh1 = Whh[0], Whh[1]
    bhh2 = bhh[None, :]
    bl2 = bl[None, :]

    deg_k = _make_degree(n, n_edges)
    spmm_x = _make_spmm(n, n_edges, T, C)
    spmm_h = _make_spmm(n, n_edges, 1, hid)

    zeros_deg = jnp.zeros((-(-n // (NS * 128)) * 128,), jnp.float32)
    zeros_x = jnp.zeros((n // NS, C), jnp.float32)
    zeros_h = jnp.zeros((n // NS, hid), jnp.float32)

    deg_parts = deg_k(src, zeros_deg).T

    outs = []
    for b in range(B):
        x = x_seq[b]
        xs = _tc_prep(x, deg_parts)
        sx = spmm_x(*[xs[t] for t in range(T)], src, dst, zeros_x)
        h, hs = _tc_t0init(x, sx, deg_parts, w0, w1, bc,
                           bhz[None, :], bhh2, hid)
        for t in range(1, T):
            sh = spmm_h(hs, src, dst, zeros_h)
            z, hr, hrs, xhp = _tc_gates(t, x, sx, sh, deg_parts, h,
                                        w0, w1, bc, wzr0, wzr1, bzr, hid)
            shr = spmm_h(hrs, src, dst, zeros_h)
            final = t == T - 1
            if final:
                h, hs, out_b = _tc_update(shr, deg_parts, h, z, hr, xhp,
                                          whh0, whh1, bhh2, hid, True,
                                          wl=Wl, bl=bl2, out_ch=out_ch)
            else:
                h, hs = _tc_update(shr, deg_parts, h, z, hr, xhp,
                                   whh0, whh1, bhh2, hid, False)
        outs.append(out_b)
    return jnp.stack(outs, axis=0)
